# baseline (device time: 22128 ns/iter reference)
import jax
import jax.numpy as jnp
from jax import lax
from jax.experimental import pallas as pl
from jax.experimental.pallas import tpu as pltpu


def kernel(x, router, W1, W2):
    t_loc, d = x.shape
    e_loc, _, f = W1.shape
    e_tot = 2 * e_loc
    half = t_loc // 2

    def body(x_ref, r_ref, w1_ref, w2_ref, out_ref,
             w1v, w2v, xsend, xpeer, rpeer, wsend, wrecv, psend, pcomb,
             local_sems, send_sems, recv_sems):
        my_x = lax.axis_index("x")
        my_y = lax.axis_index("y")
        my_z = lax.axis_index("z")
        peer = (1 - my_x, my_y, my_z)

        barrier = pltpu.get_barrier_semaphore()
        pl.semaphore_signal(barrier, inc=1, device_id=peer,
                            device_id_type=pl.DeviceIdType.MESH)

        cps = []
        for j in range(e_loc):
            c1 = pltpu.make_async_copy(
                w1_ref.at[j], w1v.at[:, pl.ds(j * f, f)], local_sems.at[j])
            c2 = pltpu.make_async_copy(
                w2_ref.at[j], w2v.at[pl.ds(j * f, f), :],
                local_sems.at[e_loc + j])
            c1.start()
            c2.start()
            cps += [c1, c2]

        xl = x_ref[:, :].astype(jnp.bfloat16)
        xsend[:, :] = xl
        g_self = jnp.dot(x_ref[:, :], r_ref[:, :],
                         preferred_element_type=jnp.float32)

        pl.semaphore_wait(barrier, 1)

        def rdma(src, dst, i):
            return pltpu.make_async_remote_copy(
                src_ref=src, dst_ref=dst,
                send_sem=send_sems.at[i], recv_sem=recv_sems.at[i],
                device_id=peer, device_id_type=pl.DeviceIdType.MESH)

        rdma_r = rdma(r_ref, rpeer, 0)
        rdma_x1 = rdma(xsend.at[pl.ds(0, half)], xpeer.at[pl.ds(0, half)], 1)
        rdma_r.start()
        rdma_x1.start()
        rdma_r.wait()

        g_peer = jnp.dot(x_ref[:, :], rpeer[:, :],
                         preferred_element_type=jnp.float32)
        g = jnp.concatenate([g_self, g_peer], axis=1)
        col = lax.broadcasted_iota(jnp.int32, g.shape, 1)
        a1 = jnp.argmax(g, axis=1)[:, None]
        oh1 = (col == a1).astype(jnp.float32)
        m1 = jnp.max(g, axis=1, keepdims=True)
        gmask = jnp.where(col == a1, -jnp.inf, g)
        a2 = jnp.argmax(gmask, axis=1)[:, None]
        oh2 = (col == a2).astype(jnp.float32)
        m2 = jnp.max(gmask, axis=1, keepdims=True)
        b = jnp.exp(m2 - m1)
        wgt = (oh1 + oh2 * b) / (1.0 + b)
        wsend[:, :] = wgt

        rdma_w = rdma(wsend, wrecv, 2)
        rdma_x2 = rdma(xsend.at[pl.ds(half, half)],
                       xpeer.at[pl.ds(half, half)], 3)
        rdma_w.start()
        rdma_x2.start()

        def experts(xb, w):
            h = jnp.maximum(
                jnp.dot(xb, w1v[:, :], preferred_element_type=jnp.float32),
                0.0)
            n = xb.shape[0]
            ws = jnp.concatenate(
                [jnp.broadcast_to(w[:, j:j + 1], (n, f))
                 for j in range(e_loc)], axis=1)
            hs = (h * ws).astype(jnp.bfloat16)
            return jnp.dot(hs, w2v[:, :], preferred_element_type=jnp.float32)

        for cp in cps:
            cp.wait()
        acc = experts(xl, wgt[:, :e_loc])

        rdma_x1.wait()
        rdma_w.wait()
        wp = wrecv[:, e_loc:e_tot]

        psend[pl.ds(0, half), :] = experts(
            xpeer[0:half, :], wp[0:half, :]).astype(jnp.bfloat16)
        rdma_p1 = rdma(psend.at[pl.ds(0, half)], pcomb.at[pl.ds(0, half)], 4)
        rdma_p1.start()

        rdma_x2.wait()
        psend[pl.ds(half, half), :] = experts(
            xpeer[half:t_loc, :], wp[half:t_loc, :]).astype(jnp.bfloat16)
        rdma_p2 = rdma(psend.at[pl.ds(half, half)],
                       pcomb.at[pl.ds(half, half)], 5)
        rdma_p2.start()

        rdma_p1.wait()
        out_ref[pl.ds(0, half), :] = (
            acc[:half] + pcomb[0:half, :].astype(jnp.float32))
        rdma_p2.wait()
        out_ref[pl.ds(half, half), :] = (
            acc[half:] + pcomb[half:t_loc, :].astype(jnp.float32))

    w1b = W1.astype(jnp.bfloat16)
    w2b = W2.astype(jnp.bfloat16)

    return pl.pallas_call(
        body,
        out_shape=jax.ShapeDtypeStruct((t_loc, d), jnp.float32),
        in_specs=[
            pl.BlockSpec(memory_space=pltpu.VMEM),
            pl.BlockSpec(memory_space=pltpu.VMEM),
            pl.BlockSpec(memory_space=pl.ANY),
            pl.BlockSpec(memory_space=pl.ANY),
        ],
        out_specs=pl.BlockSpec(memory_space=pltpu.VMEM),
        scratch_shapes=[
            pltpu.VMEM((d, e_loc * f), jnp.bfloat16),
            pltpu.VMEM((e_loc * f, d), jnp.bfloat16),
            pltpu.VMEM((t_loc, d), jnp.bfloat16),
            pltpu.VMEM((t_loc, d), jnp.bfloat16),
            pltpu.VMEM((d, e_loc), jnp.float32),
            pltpu.VMEM((t_loc, e_tot), jnp.float32),
            pltpu.VMEM((t_loc, e_tot), jnp.float32),
            pltpu.VMEM((t_loc, d), jnp.bfloat16),
            pltpu.VMEM((t_loc, d), jnp.bfloat16),
            pltpu.SemaphoreType.DMA((4,)),
            pltpu.SemaphoreType.DMA((6,)),
            pltpu.SemaphoreType.DMA((6,)),
        ],
        compiler_params=pltpu.CompilerParams(collective_id=0),
    )(x, router, w1b, w2b)


# device time: 21919 ns/iter; 1.0095x vs baseline; 1.0095x over previous
import jax
import jax.numpy as jnp
from jax import lax
from jax.experimental import pallas as pl
from jax.experimental.pallas import tpu as pltpu


def kernel(x, router, W1, W2):
    t_loc, d = x.shape
    e_loc, _, f = W1.shape
    e_tot = 2 * e_loc
    half = t_loc // 2

    def body(x_ref, r_ref, w1_ref, w2_ref, out_ref,
             w1v, w2v, xsend, xpeer, rpeer, wsend, wrecv, psend, pcomb,
             local_sems, send_sems, recv_sems):
        my_x = lax.axis_index("x")
        my_y = lax.axis_index("y")
        my_z = lax.axis_index("z")
        peer = (1 - my_x, my_y, my_z)

        barrier = pltpu.get_barrier_semaphore()
        pl.semaphore_signal(barrier, inc=1, device_id=peer,
                            device_id_type=pl.DeviceIdType.MESH)

        cp1 = pltpu.make_async_copy(w1_ref, w1v, local_sems.at[0])
        cp2 = pltpu.make_async_copy(w2_ref, w2v, local_sems.at[1])
        cp1.start()
        cp2.start()

        xl = x_ref[:, :].astype(jnp.bfloat16)
        xsend[:, :] = xl
        g_self = jnp.dot(x_ref[:, :], r_ref[:, :],
                         preferred_element_type=jnp.float32)

        pl.semaphore_wait(barrier, 1)

        def rdma(src, dst, i):
            return pltpu.make_async_remote_copy(
                src_ref=src, dst_ref=dst,
                send_sem=send_sems.at[i], recv_sem=recv_sems.at[i],
                device_id=peer, device_id_type=pl.DeviceIdType.MESH)

        rdma_r = rdma(r_ref, rpeer, 0)
        rdma_x1 = rdma(xsend.at[pl.ds(0, half)], xpeer.at[pl.ds(0, half)], 1)
        rdma_r.start()
        rdma_x1.start()
        rdma_r.wait()

        g_peer = jnp.dot(x_ref[:, :], rpeer[:, :],
                         preferred_element_type=jnp.float32)
        g = jnp.concatenate([g_self, g_peer], axis=1)
        col = lax.broadcasted_iota(jnp.int32, g.shape, 1)
        a1 = jnp.argmax(g, axis=1)[:, None]
        oh1 = (col == a1).astype(jnp.float32)
        m1 = jnp.max(g, axis=1, keepdims=True)
        gmask = jnp.where(col == a1, -jnp.inf, g)
        a2 = jnp.argmax(gmask, axis=1)[:, None]
        oh2 = (col == a2).astype(jnp.float32)
        m2 = jnp.max(gmask, axis=1, keepdims=True)
        b = jnp.exp(m2 - m1)
        wgt = (oh1 + oh2 * b) / (1.0 + b)
        wsend[:, :] = wgt

        rdma_w = rdma(wsend, wrecv, 2)
        rdma_x2 = rdma(xsend.at[pl.ds(half, half)],
                       xpeer.at[pl.ds(half, half)], 3)
        rdma_w.start()
        rdma_x2.start()

        def experts(xb, w):
            acc = jnp.zeros((xb.shape[0], d), jnp.float32)
            for j in range(e_loc):
                h = jnp.maximum(
                    jnp.dot(xb, w1v[j], preferred_element_type=jnp.float32),
                    0.0).astype(jnp.bfloat16)
                acc = acc + jnp.dot(
                    h, w2v[j], preferred_element_type=jnp.float32
                ) * w[:, j:j + 1]
            return acc

        cp1.wait()
        cp2.wait()
        acc = experts(xl, wgt[:, :e_loc])

        rdma_x1.wait()
        rdma_w.wait()
        wp = wrecv[:, e_loc:e_tot]

        psend[pl.ds(0, half), :] = experts(
            xpeer[0:half, :], wp[0:half, :]).astype(jnp.bfloat16)
        rdma_p1 = rdma(psend.at[pl.ds(0, half)], pcomb.at[pl.ds(0, half)], 4)
        rdma_p1.start()

        rdma_x2.wait()
        psend[pl.ds(half, half), :] = experts(
            xpeer[half:t_loc, :], wp[half:t_loc, :]).astype(jnp.bfloat16)
        rdma_p2 = rdma(psend.at[pl.ds(half, half)],
                       pcomb.at[pl.ds(half, half)], 5)
        rdma_p2.start()

        rdma_p1.wait()
        out_ref[pl.ds(0, half), :] = (
            acc[:half] + pcomb[0:half, :].astype(jnp.float32))
        rdma_p2.wait()
        out_ref[pl.ds(half, half), :] = (
            acc[half:] + pcomb[half:t_loc, :].astype(jnp.float32))

    w1b = W1.astype(jnp.bfloat16)
    w2b = W2.astype(jnp.bfloat16)

    return pl.pallas_call(
        body,
        out_shape=jax.ShapeDtypeStruct((t_loc, d), jnp.float32),
        in_specs=[
            pl.BlockSpec(memory_space=pltpu.VMEM),
            pl.BlockSpec(memory_space=pltpu.VMEM),
            pl.BlockSpec(memory_space=pl.ANY),
            pl.BlockSpec(memory_space=pl.ANY),
        ],
        out_specs=pl.BlockSpec(memory_space=pltpu.VMEM),
        scratch_shapes=[
            pltpu.VMEM((e_loc, d, f), jnp.bfloat16),
            pltpu.VMEM((e_loc, f, d), jnp.bfloat16),
            pltpu.VMEM((t_loc, d), jnp.bfloat16),
            pltpu.VMEM((t_loc, d), jnp.bfloat16),
            pltpu.VMEM((d, e_loc), jnp.float32),
            pltpu.VMEM((t_loc, e_tot), jnp.float32),
            pltpu.VMEM((t_loc, e_tot), jnp.float32),
            pltpu.VMEM((t_loc, d), jnp.bfloat16),
            pltpu.VMEM((t_loc, d), jnp.bfloat16),
            pltpu.SemaphoreType.DMA((2,)),
            pltpu.SemaphoreType.DMA((6,)),
            pltpu.SemaphoreType.DMA((6,)),
        ],
        compiler_params=pltpu.CompilerParams(collective_id=0),
    )(x, router, w1b, w2b)
